# baseline (speedup 1.0000x reference)
import jax
import jax.numpy as jnp
from jax import lax
from jax.experimental import pallas as pl
from jax.experimental.pallas import tpu as pltpu
from jax.experimental.pallas import tpu_sc as plsc
import functools

NC = 2
NS = 16
NW = NC * NS
LANES = 16
CH = 8
NB = 6
G = 4


@jax.jit
def _embed_sc(ids_w, token_embed, pos_embed):
    V, D = token_embed.shape
    T = pos_embed.shape[0]
    B = ids_w.shape[1] * NW // T
    t_per_w = T // NW
    nch = t_per_w // CH
    vecs_per_row = D // LANES
    vecs = CH * vecs_per_row

    mesh = plsc.VectorSubcoreMesh(
        core_axis_name="c", subcore_axis_name="s", num_cores=NC,
        num_subcores=NS)

    @functools.partial(
        pl.kernel,
        out_type=jax.ShapeDtypeStruct((B * T, D), jnp.float32),
        mesh=mesh,
        scratch_types=(
            [pltpu.VMEM((B * t_per_w,), jnp.int32),
             pltpu.VMEM((t_per_w, D), jnp.float32)]
            + [pltpu.VMEM((CH, D), jnp.float32)] * NB
            + [pltpu.SemaphoreType.DMA] * (2 * NB + 2)
        ),
    )
    def k(ids_hbm, tok_hbm, pos_hbm, out_hbm, idx_v, pos_v, *scr):
        bufs = scr[:NB]
        gsem = scr[NB:2 * NB]
        ssem = scr[2 * NB:3 * NB]
        psem, isem = scr[3 * NB], scr[3 * NB + 1]
        wid = lax.axis_index("s") * NC + lax.axis_index("c")
        t0 = wid * t_per_w
        idx_d = pltpu.async_copy(ids_hbm.at[wid], idx_v, isem)
        pos_d = pltpu.async_copy(pos_hbm.at[pl.ds(t0, t_per_w)], pos_v,
                                 psem)

        chunks = [(b, h) for b in range(B) for h in range(nch)]
        n = len(chunks)
        gd = [None] * n
        sd = [None] * n

        def start_gather(i):
            b, h = chunks[i]
            idx_slice = idx_v.at[pl.ds((b * nch + h) * CH, CH)]
            gd[i] = pltpu.async_copy(tok_hbm.at[idx_slice], bufs[i % NB],
                                     gsem[i % NB])

        def add_pos_half(i, half):
            _, h = chunks[i]
            buf = bufs[i % NB]
            p_base = h * CH
            r0 = half * (CH // 2)

            @plsc.parallel_loop(0, vecs // 2, unroll=16)
            def _(v):
                row = r0 + v // vecs_per_row
                col = (v % vecs_per_row) * LANES
                x = pos_v[p_base + row, pl.ds(col, LANES)]
                plsc.addupdate(buf.at[row, pl.ds(col, LANES)], x)

        def start_store_half(i, half):
            b, h = chunks[i]
            r0 = half * (CH // 2)
            d = pltpu.async_copy(
                bufs[i % NB].at[pl.ds(r0, CH // 2)],
                out_hbm.at[pl.ds(b * T + t0 + h * CH + r0, CH // 2)],
                ssem[i % NB])
            if half == 0:
                sd[i] = [d]
            else:
                sd[i].append(d)

        idx_d.wait()
        for i in range(G):
            start_gather(i)
        pos_d.wait()
        for i in range(n):
            gd[i].wait()
            add_pos_half(i, 0)
            start_store_half(i, 0)
            add_pos_half(i, 1)
            start_store_half(i, 1)
            j = i + G
            if j < n:
                if j - NB >= 0:
                    for d in sd[j - NB]:
                        d.wait()
                start_gather(j)
        for i in range(max(0, n - NB), n):
            for d in sd[i]:
                d.wait()

    return k(ids_w, token_embed, pos_embed)


def kernel(input_ids, token_embed, pos_embed):
    B, T = input_ids.shape
    D = token_embed.shape[1]
    t_per_w = T // NW
    ids_w = (input_ids.astype(jnp.int32)
             .reshape(B, NW, t_per_w)
             .transpose(1, 0, 2)
             .reshape(NW, B * t_per_w))
    out = _embed_sc(ids_w, token_embed, pos_embed[:T])
    return out.reshape(B, T, D)


# speedup vs baseline: 1.0950x; 1.0950x over previous
import jax
import jax.numpy as jnp
from jax import lax
from jax.experimental import pallas as pl
from jax.experimental.pallas import tpu as pltpu
from jax.experimental.pallas import tpu_sc as plsc
import functools

NC = 2
NS = 16
NW = NC * NS
LANES = 16
CH = 8
NB = 6
G = 4


@jax.jit
def _embed_sc(ids_w, token_embed, pos_embed):
    V, D = token_embed.shape
    T = pos_embed.shape[0]
    B = ids_w.shape[1] * NW // T
    t_per_w = T // NW
    nch = t_per_w // CH
    vecs_per_row = D // LANES
    vecs = CH * vecs_per_row

    mesh = plsc.VectorSubcoreMesh(
        core_axis_name="c", subcore_axis_name="s", num_cores=NC,
        num_subcores=NS)

    @functools.partial(
        pl.kernel,
        out_type=jax.ShapeDtypeStruct((B * T, D), jnp.float32),
        mesh=mesh,
        scratch_types=(
            [pltpu.VMEM((B * t_per_w,), jnp.int32),
             pltpu.VMEM((t_per_w, D), jnp.float32)]
            + [pltpu.VMEM((CH, D), jnp.float32)] * NB
            + [pltpu.SemaphoreType.DMA] * (2 * NB + 2)
        ),
    )
    def k(ids_hbm, tok_hbm, pos_hbm, out_hbm, idx_v, pos_v, *scr):
        bufs = scr[:NB]
        gsem = scr[NB:2 * NB]
        ssem = scr[2 * NB:3 * NB]
        psem, isem = scr[3 * NB], scr[3 * NB + 1]
        wid = lax.axis_index("s") * NC + lax.axis_index("c")
        t0 = wid * t_per_w
        idx_d = pltpu.async_copy(ids_hbm.at[wid], idx_v, isem)
        pos_d = pltpu.async_copy(pos_hbm.at[pl.ds(t0, t_per_w)], pos_v,
                                 psem)

        chunks = [(b, h) for b in range(B) for h in range(nch)]
        n = len(chunks)
        gd = [None] * n
        sd = [None] * n

        def start_gather(i):
            b, h = chunks[i]
            idx_slice = idx_v.at[pl.ds((b * nch + h) * CH, CH)]
            gd[i] = pltpu.async_copy(tok_hbm.at[idx_slice], bufs[i % NB],
                                     gsem[i % NB])

        def add_pos(i):
            _, h = chunks[i]
            buf = bufs[i % NB]
            p_base = h * CH

            @plsc.parallel_loop(0, vecs, unroll=16)
            def _(v):
                row = v // vecs_per_row
                col = (v % vecs_per_row) * LANES
                x = pos_v[p_base + row, pl.ds(col, LANES)]
                plsc.addupdate(buf.at[row, pl.ds(col, LANES)], x)

        def start_store(i):
            b, h = chunks[i]
            sd[i] = pltpu.async_copy(
                bufs[i % NB], out_hbm.at[pl.ds(b * T + t0 + h * CH, CH)],
                ssem[i % NB])

        idx_d.wait()
        for i in range(G):
            start_gather(i)
        pos_d.wait()
        for i in range(n):
            j = i + G
            if j < n:
                if j - NB >= 0:
                    sd[j - NB].wait()
                start_gather(j)
            gd[i].wait()
            add_pos(i)
            start_store(i)
        for i in range(max(0, n - NB), n):
            sd[i].wait()

    return k(ids_w, token_embed, pos_embed)


def kernel(input_ids, token_embed, pos_embed):
    B, T = input_ids.shape
    D = token_embed.shape[1]
    t_per_w = T // NW
    ids_w = (input_ids.astype(jnp.int32)
             .reshape(B, NW, t_per_w)
             .transpose(1, 0, 2)
             .reshape(NW, B * t_per_w))
    out = _embed_sc(ids_w, token_embed, pos_embed[:T])
    return out.reshape(B, T, D)
